# baseline (speedup 1.0000x reference)
import functools

import jax
import jax.numpy as jnp
from jax import lax
from jax.experimental import pallas as pl
from jax.experimental.pallas import tpu as pltpu
from jax.experimental.pallas import tpu_sc as plsc

_BN = 4096


def _dense_body(parts_ref, x_ref, wl_ref, wr_ref, bl_ref,
                W1_ref, b1_ref, W2_ref, b2_ref, o_ref):
    p = parts_ref[...]
    d = jnp.maximum(p[0:1, :] + p[1:2, :], 1.0)
    hT = wl_ref[...] * d + wr_ref[...] * x_ref[...] + bl_ref[...]
    hT = jnp.maximum(hT, 0.0).astype(jnp.bfloat16)
    h1 = lax.dot_general(hT, W1_ref[...], (((0,), (1,)), ((), ())),
                         preferred_element_type=jnp.float32) + b1_ref[...]
    h1 = jnp.maximum(h1, 0.0).astype(jnp.bfloat16)
    o_ref[...] = lax.dot_general(h1, W2_ref[...], (((1,), (1,)), ((), ())),
                                 preferred_element_type=jnp.float32) + b2_ref[...]


def _dense(parts, x_row, W_l0, b_l0, W_r0, W_r1, b_r1, W_r2, b_r2,
           interpret=False):
    n = x_row.shape[1]
    n_pad = parts.shape[1]
    H = W_r1.shape[0]
    lane = pl.BlockSpec((1, _BN), lambda i: (0, i))
    colw = pl.BlockSpec((H, 1), lambda i: (0, 0))
    roww = pl.BlockSpec((1, H), lambda i: (0, 0))
    mat = pl.BlockSpec((H, H), lambda i: (0, 0))
    return pl.pallas_call(
        _dense_body,
        grid=(n_pad // _BN,),
        in_specs=[pl.BlockSpec((2, _BN), lambda i: (0, i)), lane,
                  colw, colw, colw, mat, roww, mat, roww],
        out_specs=pl.BlockSpec((_BN, H), lambda i: (i, 0)),
        out_shape=jax.ShapeDtypeStruct((n, H), jnp.float32),
        interpret=interpret,
    )(parts, x_row,
      W_l0, W_r0, b_l0.reshape(H, 1),
      W_r1.astype(jnp.bfloat16), b_r1.reshape(1, H),
      W_r2.astype(jnp.bfloat16), b_r2.reshape(1, H))


def _make_deg_kernel(n_pad, half_rows, trash, nc, ns, nw):
    k_full = ((-(-half_rows // nw) + 7) // 8) * 8
    rows_last = half_rows - (nw - 1) * k_full
    fetch_full = 2 * k_full
    fetch_last = 2 * rows_last
    k_buf = fetch_full
    per_slice = n_pad // ns
    grp = 28 if k_full % 28 == 0 else 8
    mesh = plsc.VectorSubcoreMesh(core_axis_name="c", subcore_axis_name="s")

    @functools.partial(
        pl.kernel, mesh=mesh,
        out_type=jax.ShapeDtypeStruct((nc, n_pad), jnp.float32),
        scratch_types=[
            pltpu.VMEM((k_buf, 128), jnp.int32),
            pltpu.VMEM((128,), jnp.float32),
            pltpu.VMEM((per_slice,), jnp.float32),
            pltpu.VMEM_SHARED((n_pad,), jnp.float32),
            pltpu.SemaphoreType.DMA,
        ])
    def deg_kernel(edges_hbm, out_hbm, idx_v, ones_v, buf_v, hist_sh, sem):
        c = lax.axis_index("c")
        s = lax.axis_index("s")
        wid = s * nc + c

        def fill_ones(i, _):
            ones_v[pl.ds(i * 16, 16)] = jnp.full((16,), 1.0, jnp.float32)
            return 0
        lax.fori_loop(0, 128 // 16, fill_ones, 0)

        @pl.when(wid == nw - 1)
        def _():
            for r in range(2 * rows_last + 1, 2 * k_full, 2):
                for l in range(128 // 16):
                    idx_v[r, pl.ds(l * 16, 16)] = (
                        jnp.full((16,), trash, jnp.int32))

        def fill_zero(i, _):
            buf_v[pl.ds(i * 16, 16)] = jnp.zeros((16,), jnp.float32)
            return 0
        lax.fori_loop(0, per_slice // 16, fill_zero, 0)
        pltpu.sync_copy(buf_v, hist_sh.at[pl.ds(s * per_slice, per_slice)])

        abase = wid * fetch_full

        @pl.when(wid < nw - 1)
        def _():
            pltpu.sync_copy(edges_hbm.at[pl.ds(abase, fetch_full)],
                            idx_v.at[pl.ds(0, fetch_full)])

        @pl.when(wid == nw - 1)
        def _():
            pltpu.sync_copy(edges_hbm.at[pl.ds(abase, fetch_last)],
                            idx_v.at[pl.ds(0, fetch_last)])

        plsc.subcore_barrier()

        def group(g, _):
            copies = [
                pltpu.async_copy(ones_v,
                                 hist_sh.at[idx_v.at[2 * (g * grp + b) + 1]],
                                 sem, add=True)
                for b in range(grp)
            ]
            for cp in copies:
                cp.wait()
            return 0
        lax.fori_loop(0, k_full // grp, group, 0)
        plsc.subcore_barrier()
        pltpu.sync_copy(hist_sh.at[pl.ds(s * per_slice, per_slice)], buf_v)
        pltpu.sync_copy(buf_v, out_hbm.at[c, pl.ds(s * per_slice, per_slice)])

    return deg_kernel


def _deg_parts(adjs, n, n_pad):
    info = plsc.get_sparse_core_info()
    nc, ns = info.num_cores, info.num_subcores
    e = adjs.shape[1]
    inter = jnp.transpose(
        adjs.astype(jnp.int32).reshape(2, e // 128, 128),
        (1, 0, 2)).reshape(2 * e // 128, 128)
    return _make_deg_kernel(n_pad, e // 128, n, nc, ns, nc * ns)(inter)


def kernel(x, adjs, W_l0, b_l0, W_r0, W_r1, b_r1, W_r2, b_r2):
    n = x.shape[0]
    n_pad = ((n + _BN - 1) // _BN) * _BN
    parts = _deg_parts(adjs, n, n_pad)
    return _dense(parts, x.reshape(1, n),
                  W_l0, b_l0, W_r0, W_r1, b_r1, W_r2, b_r2)


# speedup vs baseline: 1.0817x; 1.0817x over previous
import functools

import jax
import jax.numpy as jnp
from jax import lax
from jax.experimental import pallas as pl
from jax.experimental.pallas import tpu as pltpu
from jax.experimental.pallas import tpu_sc as plsc

_BN = 4096


def _dense_body(parts_ref, x_ref, wl_ref, wr_ref, bl_ref,
                W1_ref, b1_ref, W2_ref, b2_ref, o_ref):
    p = parts_ref[...]
    d = jnp.maximum(p[0:1, :] + p[1:2, :], 1.0)
    hT = wl_ref[...] * d + wr_ref[...] * x_ref[...] + bl_ref[...]
    hT = jnp.maximum(hT, 0.0).astype(jnp.bfloat16)
    h1 = lax.dot_general(hT, W1_ref[...], (((0,), (1,)), ((), ())),
                         preferred_element_type=jnp.float32) + b1_ref[...]
    h1 = jnp.maximum(h1, 0.0).astype(jnp.bfloat16)
    o_ref[...] = lax.dot_general(h1, W2_ref[...], (((1,), (1,)), ((), ())),
                                 preferred_element_type=jnp.float32) + b2_ref[...]


def _dense(parts, x_row, W_l0, b_l0, W_r0, W_r1, b_r1, W_r2, b_r2,
           interpret=False):
    n = x_row.shape[1]
    n_pad = parts.shape[1]
    H = W_r1.shape[0]
    lane = pl.BlockSpec((1, _BN), lambda i: (0, i))
    colw = pl.BlockSpec((H, 1), lambda i: (0, 0))
    roww = pl.BlockSpec((1, H), lambda i: (0, 0))
    mat = pl.BlockSpec((H, H), lambda i: (0, 0))
    return pl.pallas_call(
        _dense_body,
        grid=(n_pad // _BN,),
        in_specs=[pl.BlockSpec((2, _BN), lambda i: (0, i)), lane,
                  colw, colw, colw, mat, roww, mat, roww],
        out_specs=pl.BlockSpec((_BN, H), lambda i: (i, 0)),
        out_shape=jax.ShapeDtypeStruct((n, H), jnp.float32),
        interpret=interpret,
    )(parts, x_row,
      W_l0, W_r0, b_l0.reshape(H, 1),
      W_r1.astype(jnp.bfloat16), b_r1.reshape(1, H),
      W_r2.astype(jnp.bfloat16), b_r2.reshape(1, H))


def _make_deg_kernel(n_pad, half_rows, trash, nc, ns, nw):
    k_full = ((-(-half_rows // nw) + 7) // 8) * 8
    rows_last = half_rows - (nw - 1) * k_full
    off = half_rows % 8
    fetch_full = ((off + k_full + 7) // 8) * 8
    fetch_last = ((off + rows_last + 7) // 8) * 8
    k_buf = fetch_full
    per_slice = n_pad // ns
    grp = 56 if k_full % 56 == 0 else 8
    mesh = plsc.VectorSubcoreMesh(core_axis_name="c", subcore_axis_name="s")

    @functools.partial(
        pl.kernel, mesh=mesh,
        out_type=jax.ShapeDtypeStruct((nc, n_pad), jnp.float32),
        scratch_types=[
            pltpu.VMEM((k_buf, 128), jnp.int32),
            pltpu.VMEM((128,), jnp.float32),
            pltpu.VMEM((per_slice,), jnp.float32),
            pltpu.VMEM_SHARED((n_pad,), jnp.float32),
            pltpu.SemaphoreType.DMA,
        ])
    def deg_kernel(edges_hbm, out_hbm, idx_v, ones_v, buf_v, hist_sh, sem):
        c = lax.axis_index("c")
        s = lax.axis_index("s")
        wid = s * nc + c

        def fill_ones(i, _):
            ones_v[pl.ds(i * 16, 16)] = jnp.full((16,), 1.0, jnp.float32)
            return 0
        lax.fori_loop(0, 128 // 16, fill_ones, 0)

        @pl.when(wid == nw - 1)
        def _():
            for r in range(fetch_last, off + k_full):
                for l in range(128 // 16):
                    idx_v[r, pl.ds(l * 16, 16)] = (
                        jnp.full((16,), trash, jnp.int32))

        def fill_zero(i, _):
            buf_v[pl.ds(i * 16, 16)] = jnp.zeros((16,), jnp.float32)
            return 0
        lax.fori_loop(0, per_slice // 16, fill_zero, 0)
        pltpu.sync_copy(buf_v, hist_sh.at[pl.ds(s * per_slice, per_slice)])

        abase = (half_rows - off) + wid * k_full

        @pl.when(wid < nw - 1)
        def _():
            pltpu.sync_copy(edges_hbm.at[pl.ds(abase, fetch_full)],
                            idx_v.at[pl.ds(0, fetch_full)])

        @pl.when(wid == nw - 1)
        def _():
            pltpu.sync_copy(edges_hbm.at[pl.ds(abase, fetch_last)],
                            idx_v.at[pl.ds(0, fetch_last)])

        plsc.subcore_barrier()

        def group(g, _):
            copies = [
                pltpu.async_copy(ones_v,
                                 hist_sh.at[idx_v.at[off + g * grp + b]],
                                 sem, add=True)
                for b in range(grp)
            ]
            for cp in copies:
                cp.wait()
            return 0
        lax.fori_loop(0, k_full // grp, group, 0)
        plsc.subcore_barrier()
        pltpu.sync_copy(hist_sh.at[pl.ds(s * per_slice, per_slice)], buf_v)
        pltpu.sync_copy(buf_v, out_hbm.at[c, pl.ds(s * per_slice, per_slice)])

    return deg_kernel


def _deg_parts(adjs, n, n_pad):
    info = plsc.get_sparse_core_info()
    nc, ns = info.num_cores, info.num_subcores
    e = adjs.shape[1]
    flat2d = adjs.astype(jnp.int32).reshape(2 * e // 128, 128)
    return _make_deg_kernel(n_pad, e // 128, n, nc, ns, nc * ns)(flat2d)


def kernel(x, adjs, W_l0, b_l0, W_r0, W_r1, b_r1, W_r2, b_r2):
    n = x.shape[0]
    n_pad = ((n + _BN - 1) // _BN) * _BN
    parts = _deg_parts(adjs, n, n_pad)
    return _dense(parts, x.reshape(1, n),
                  W_l0, b_l0, W_r0, W_r1, b_r1, W_r2, b_r2)


# speedup vs baseline: 1.1093x; 1.0256x over previous
import functools

import jax
import jax.numpy as jnp
from jax import lax
from jax.experimental import pallas as pl
from jax.experimental.pallas import tpu as pltpu
from jax.experimental.pallas import tpu_sc as plsc

_BN = 4096


def _dense_body(pa_ref, pb_ref, x_ref, wl_ref, wr_ref, bl_ref,
                W1_ref, b1_ref, W2_ref, b2_ref, o_ref):
    p = pa_ref[...] + pb_ref[...]
    d = jnp.maximum(p[0:1, :] + p[1:2, :], 1.0)
    hT = wl_ref[...] * d + wr_ref[...] * x_ref[...] + bl_ref[...]
    hT = jnp.maximum(hT, 0.0).astype(jnp.bfloat16)
    h1 = lax.dot_general(hT, W1_ref[...], (((0,), (1,)), ((), ())),
                         preferred_element_type=jnp.float32) + b1_ref[...]
    h1 = jnp.maximum(h1, 0.0).astype(jnp.bfloat16)
    o_ref[...] = lax.dot_general(h1, W2_ref[...], (((1,), (1,)), ((), ())),
                                 preferred_element_type=jnp.float32) + b2_ref[...]


def _dense(parts_a, parts_b, x_row, W_l0, b_l0, W_r0, W_r1, b_r1, W_r2, b_r2,
           interpret=False):
    n = x_row.shape[1]
    n_pad = parts_a.shape[1]
    H = W_r1.shape[0]
    pair = pl.BlockSpec((2, _BN), lambda i: (0, i))
    lane = pl.BlockSpec((1, _BN), lambda i: (0, i))
    colw = pl.BlockSpec((H, 1), lambda i: (0, 0))
    roww = pl.BlockSpec((1, H), lambda i: (0, 0))
    mat = pl.BlockSpec((H, H), lambda i: (0, 0))
    return pl.pallas_call(
        _dense_body,
        grid=(n_pad // _BN,),
        in_specs=[pair, pair, lane,
                  colw, colw, colw, mat, roww, mat, roww],
        out_specs=pl.BlockSpec((_BN, H), lambda i: (i, 0)),
        out_shape=jax.ShapeDtypeStruct((n, H), jnp.float32),
        interpret=interpret,
    )(parts_a, parts_b, x_row,
      W_l0, W_r0, b_l0.reshape(H, 1),
      W_r1.astype(jnp.bfloat16), b_r1.reshape(1, H),
      W_r2.astype(jnp.bfloat16), b_r2.reshape(1, H))


def _make_deg_kernel(n_pad, half_rows, trash, nc, ns, nw):
    k_full = ((-(-half_rows // nw) + 7) // 8) * 8
    rows_last = half_rows - (nw - 1) * k_full
    off = half_rows % 8
    fetch_full = ((off + k_full + 7) // 8) * 8
    fetch_last = ((off + rows_last + 7) // 8) * 8
    assert fetch_last == off + rows_last
    k_buf = fetch_full
    per_slice = n_pad // ns
    grp = next(g for g in (28, 14, 8, 4) if k_full % g == 0)
    rows_pad_last = -(-rows_last // grp) * grp
    mesh = plsc.VectorSubcoreMesh(core_axis_name="c", subcore_axis_name="s")

    @functools.partial(
        pl.kernel, mesh=mesh,
        out_type=jax.ShapeDtypeStruct((nc, n_pad), jnp.float32),
        scratch_types=[
            pltpu.VMEM((k_buf, 128), jnp.int32),
            pltpu.VMEM((128,), jnp.float32),
            pltpu.VMEM((per_slice,), jnp.float32),
            pltpu.VMEM_SHARED((n_pad,), jnp.float32),
            pltpu.SemaphoreType.DMA,
        ])
    def deg_kernel(edges_hbm, out_hbm, idx_v, ones_v, buf_v, hist_sh, sem):
        c = lax.axis_index("c")
        s = lax.axis_index("s")
        wid = s * nc + c

        def fill_ones(i, _):
            ones_v[pl.ds(i * 16, 16)] = jnp.full((16,), 1.0, jnp.float32)
            return 0
        lax.fori_loop(0, 128 // 16, fill_ones, 0)

        @pl.when(wid == nw - 1)
        def _():
            for r in range(fetch_last, off + rows_pad_last):
                for l in range(128 // 16):
                    idx_v[r, pl.ds(l * 16, 16)] = (
                        jnp.full((16,), trash, jnp.int32))

        def fill_zero(i, _):
            buf_v[pl.ds(i * 16, 16)] = jnp.zeros((16,), jnp.float32)
            return 0
        lax.fori_loop(0, per_slice // 16, fill_zero, 0)
        pltpu.sync_copy(buf_v, hist_sh.at[pl.ds(s * per_slice, per_slice)])

        abase = (half_rows - off) + wid * k_full

        @pl.when(wid < nw - 1)
        def _():
            pltpu.sync_copy(edges_hbm.at[pl.ds(abase, fetch_full)],
                            idx_v.at[pl.ds(0, fetch_full)])

        @pl.when(wid == nw - 1)
        def _():
            pltpu.sync_copy(edges_hbm.at[pl.ds(abase, fetch_last)],
                            idx_v.at[pl.ds(0, fetch_last)])

        plsc.subcore_barrier()

        def group(g, _):
            copies = [
                pltpu.async_copy(ones_v,
                                 hist_sh.at[idx_v.at[off + g * grp + b]],
                                 sem, add=True)
                for b in range(grp)
            ]
            for cp in copies:
                cp.wait()
            return 0

        @pl.when(wid < nw - 1)
        def _():
            lax.fori_loop(0, k_full // grp, group, 0)

        @pl.when(wid == nw - 1)
        def _():
            lax.fori_loop(0, rows_pad_last // grp, group, 0)
        plsc.subcore_barrier()
        pltpu.sync_copy(hist_sh.at[pl.ds(s * per_slice, per_slice)], buf_v)
        pltpu.sync_copy(buf_v, out_hbm.at[c, pl.ds(s * per_slice, per_slice)])

    return deg_kernel


def _deg_parts(adjs, n, n_pad):
    info = plsc.get_sparse_core_info()
    nc, ns = info.num_cores, info.num_subcores
    e = adjs.shape[1]
    ea = (e // 2048) * 1024
    adjs32 = adjs.astype(jnp.int32)
    outs = []
    for lo, hi in ((0, ea), (ea, e)):
        eh = hi - lo
        flat = adjs32[:, lo:hi].reshape(2 * eh // 128, 128)
        outs.append(
            _make_deg_kernel(n_pad, eh // 128, n, nc, ns, nc * ns)(flat))
    return outs


def kernel(x, adjs, W_l0, b_l0, W_r0, W_r1, b_r1, W_r2, b_r2):
    n = x.shape[0]
    n_pad = ((n + _BN - 1) // _BN) * _BN
    parts_a, parts_b = _deg_parts(adjs, n, n_pad)
    return _dense(parts_a, parts_b, x.reshape(1, n),
                  W_l0, b_l0, W_r0, W_r1, b_r1, W_r2, b_r2)
